# baseline (speedup 1.0000x reference)
import functools

import jax
import jax.numpy as jnp
from jax import lax
from jax.experimental import pallas as pl
from jax.experimental.pallas import tpu as pltpu
from jax.experimental.pallas import tpu_sc as plsc

_D = 64
_BB = 128
_T = 200


@functools.cache
def _build(n_b: int, n_t: int, d: int):
    info = plsc.get_sparse_core_info()
    nc = info.num_cores
    nw = nc * info.num_subcores
    assert n_b == nw * _BB and n_t == _T and d == _D

    mesh = plsc.VectorSubcoreMesh(core_axis_name="c", subcore_axis_name="s")

    @functools.partial(
        pl.kernel,
        mesh=mesh,
        out_type=jax.ShapeDtypeStruct((_T, d // 8, n_b // 128, 8, 128),
                                      jnp.float32),
        compiler_params=pltpu.CompilerParams(
            use_tc_tiling_on_sc=False, needs_layout_passes=False
        ),
        scratch_types=[
            pltpu.VMEM((_T, _BB), jnp.int32),
            pltpu.VMEM((_BB, d), jnp.float32),
            pltpu.VMEM((_BB, d), jnp.float32),
            pltpu.VMEM((d, 128), jnp.float32),
            pltpu.SemaphoreType.DMA,
            pltpu.SemaphoreType.DMA,
        ],
    )
    def k(xt_hbm, table_hbm, out_hbm, idx_v, rows0, rows1, tr_v, g0, g1):
        wid = lax.axis_index("s") * nc + lax.axis_index("c")
        pltpu.sync_copy(xt_hbm.at[:, pl.ds(wid * _BB, _BB)], idx_v)

        lanes = jnp.arange(16, dtype=jnp.int32)

        def fire_g(t, rows, gsem):
            pltpu.async_copy(table_hbm.at[idx_v.at[t]], rows, gsem)

        def wait_g(rows, gsem):
            pltpu.make_async_copy(table_hbm.at[pl.ds(0, _BB)], rows, gsem).wait()

        def step(t, rows):
            def dbody(dcol, carry):
                dvec = jnp.full((16,), dcol, jnp.int32)
                for g in range(8):
                    v = plsc.load_gather(rows, [lanes + 16 * g, dvec])
                    tr_v[dcol, pl.ds(16 * g, 16)] = v
                return carry

            lax.fori_loop(0, d, dbody, 0)

            for dd in range(d // 8):
                pltpu.sync_copy(
                    tr_v.at[pl.ds(dd * 8, 8)], out_hbm.at[t, dd, wid]
                )

        fire_g(0, rows0, g0)
        fire_g(1, rows1, g1)

        def body(i, carry):
            t0 = 2 * i
            wait_g(rows0, g0)
            step(t0, rows0)
            fire_g(t0 + 2, rows0, g0)
            wait_g(rows1, g1)
            step(t0 + 1, rows1)
            fire_g(t0 + 3, rows1, g1)
            return carry

        lax.fori_loop(0, _T // 2 - 1, body, 0)

        wait_g(rows0, g0)
        step(_T - 2, rows0)
        wait_g(rows1, g1)
        step(_T - 1, rows1)

    return k


def kernel(x, table):
    s0, s1 = x.shape
    out5 = _build(s0, s1, _D)(x.T, table)
    return out5.transpose(2, 4, 0, 1, 3).reshape(s0, s1, _D)


# speedup vs baseline: 1.6182x; 1.6182x over previous
import functools

import jax
import jax.numpy as jnp
from jax import lax
from jax.experimental import pallas as pl
from jax.experimental.pallas import tpu as pltpu
from jax.experimental.pallas import tpu_sc as plsc

_D = 64
_BB = 128
_T = 200


@functools.cache
def _build(n_b: int, n_t: int, d: int):
    info = plsc.get_sparse_core_info()
    nc = info.num_cores
    nw = nc * info.num_subcores
    assert n_b == nw * _BB and n_t == _T and d == _D

    mesh = plsc.VectorSubcoreMesh(core_axis_name="c", subcore_axis_name="s")

    @functools.partial(
        pl.kernel,
        mesh=mesh,
        out_type=jax.ShapeDtypeStruct((_T, d // 8, n_b // 128, 8, 128),
                                      jnp.float32),
        compiler_params=pltpu.CompilerParams(
            use_tc_tiling_on_sc=False, needs_layout_passes=False
        ),
        scratch_types=[
            pltpu.VMEM((_T, _BB), jnp.int32),
            pltpu.VMEM((_BB, d), jnp.float32),
            pltpu.VMEM((_BB, d), jnp.float32),
            pltpu.VMEM((d, 129), jnp.float32),
            pltpu.SemaphoreType.DMA,
            pltpu.SemaphoreType.DMA,
        ],
    )
    def k(xt_hbm, table_hbm, out_hbm, idx_v, rows0, rows1, tr_v, g0, g1):
        wid = lax.axis_index("s") * nc + lax.axis_index("c")
        pltpu.sync_copy(xt_hbm.at[:, pl.ds(wid * _BB, _BB)], idx_v)

        lanes = jnp.arange(16, dtype=jnp.int32)

        def fire_g(t, rows, gsem):
            pltpu.async_copy(table_hbm.at[idx_v.at[t]], rows, gsem)

        def wait_g(rows, gsem):
            pltpu.make_async_copy(table_hbm.at[pl.ds(0, _BB)], rows, gsem).wait()

        dvecs = [lanes + 16 * g for g in range(d // 16)]

        def step(t, rows):
            for c in range(_BB):
                cvec = jnp.full((16,), c, jnp.int32)
                for g in range(d // 16):
                    v = rows[c, pl.ds(16 * g, 16)]
                    plsc.store_scatter(tr_v, [dvecs[g], cvec], v)

            for dd in range(d // 8):
                pltpu.sync_copy(
                    tr_v.at[pl.ds(dd * 8, 8), pl.ds(0, 128)],
                    out_hbm.at[t, dd, wid],
                )

        fire_g(0, rows0, g0)
        fire_g(1, rows1, g1)

        def body(i, carry):
            t0 = 2 * i
            wait_g(rows0, g0)
            step(t0, rows0)
            fire_g(t0 + 2, rows0, g0)
            wait_g(rows1, g1)
            step(t0 + 1, rows1)
            fire_g(t0 + 3, rows1, g1)
            return carry

        lax.fori_loop(0, _T // 2 - 1, body, 0)

        wait_g(rows0, g0)
        step(_T - 2, rows0)
        wait_g(rows1, g1)
        step(_T - 1, rows1)

    return k


def kernel(x, table):
    s0, s1 = x.shape
    out5 = _build(s0, s1, _D)(x.T, table)
    return out5.transpose(2, 4, 0, 1, 3).reshape(s0, s1, _D)


# speedup vs baseline: 1.7456x; 1.0787x over previous
import functools

import jax
import jax.numpy as jnp
from jax import lax
from jax.experimental import pallas as pl
from jax.experimental.pallas import tpu as pltpu
from jax.experimental.pallas import tpu_sc as plsc

_D = 64
_BB = 128
_T = 200


@functools.cache
def _build(n_b: int, n_t: int, d: int):
    info = plsc.get_sparse_core_info()
    nc = info.num_cores
    nw = nc * info.num_subcores
    assert n_b == nw * _BB and n_t == _T and d == _D

    mesh = plsc.VectorSubcoreMesh(core_axis_name="c", subcore_axis_name="s")

    @functools.partial(
        pl.kernel,
        mesh=mesh,
        out_type=jax.ShapeDtypeStruct((_T, d // 8, n_b // 128, 8, 128),
                                      jnp.float32),
        compiler_params=pltpu.CompilerParams(
            use_tc_tiling_on_sc=False, needs_layout_passes=False
        ),
        scratch_types=[
            pltpu.VMEM((_T, _BB), jnp.int32),
            pltpu.VMEM((_BB, d), jnp.float32),
            pltpu.VMEM((_BB, d), jnp.float32),
            pltpu.VMEM((d, 129), jnp.float32),
            pltpu.VMEM((d, 129), jnp.float32),
            pltpu.SemaphoreType.DMA,
            pltpu.SemaphoreType.DMA,
            pltpu.SemaphoreType.DMA,
            pltpu.SemaphoreType.DMA,
        ],
    )
    def k(xt_hbm, table_hbm, out_hbm, idx_v, rows0, rows1, tr0, tr1,
          g0, g1, o0, o1):
        wid = lax.axis_index("s") * nc + lax.axis_index("c")
        pltpu.sync_copy(xt_hbm.at[:, pl.ds(wid * _BB, _BB)], idx_v)

        lanes = jnp.arange(16, dtype=jnp.int32)

        def fire_g(t, rows, gsem):
            pltpu.async_copy(table_hbm.at[idx_v.at[t]], rows, gsem)

        def wait_g(rows, gsem):
            pltpu.make_async_copy(table_hbm.at[pl.ds(0, _BB)], rows, gsem).wait()

        dvecs = [lanes + 16 * g for g in range(d // 16)]

        def transpose(rows, tr):
            for c in range(_BB):
                cvec = jnp.full((16,), c, jnp.int32)
                for g in range(d // 16):
                    v = rows[c, pl.ds(16 * g, 16)]
                    plsc.store_scatter(tr, [dvecs[g], cvec], v)

        def fire_o(t, tr, osem):
            for dd in range(d // 8):
                pltpu.async_copy(
                    tr.at[pl.ds(dd * 8, 8), pl.ds(0, 128)],
                    out_hbm.at[t, dd, wid],
                    osem,
                )

        def wait_o(tr, osem):
            for dd in range(d // 8):
                pltpu.make_async_copy(
                    tr.at[pl.ds(dd * 8, 8), pl.ds(0, 128)],
                    out_hbm.at[0, dd, wid],
                    osem,
                ).wait()

        fire_g(0, rows0, g0)
        fire_g(1, rows1, g1)

        def body(i, carry):
            t0 = 2 * i

            wait_g(rows0, g0)

            @pl.when(i > 0)
            def _():
                wait_o(tr0, o0)

            transpose(rows0, tr0)
            fire_g(t0 + 2, rows0, g0)
            fire_o(t0, tr0, o0)

            wait_g(rows1, g1)

            @pl.when(i > 0)
            def _():
                wait_o(tr1, o1)

            transpose(rows1, tr1)
            fire_g(t0 + 3, rows1, g1)
            fire_o(t0 + 1, tr1, o1)
            return carry

        lax.fori_loop(0, _T // 2 - 1, body, 0)

        wait_g(rows0, g0)
        wait_o(tr0, o0)
        transpose(rows0, tr0)
        fire_o(_T - 2, tr0, o0)
        wait_g(rows1, g1)
        wait_o(tr1, o1)
        transpose(rows1, tr1)
        fire_o(_T - 1, tr1, o1)
        wait_o(tr0, o0)
        wait_o(tr1, o1)

    return k


def kernel(x, table):
    s0, s1 = x.shape
    out5 = _build(s0, s1, _D)(x.T, table)
    return out5.transpose(2, 4, 0, 1, 3).reshape(s0, s1, _D)


# speedup vs baseline: 1.9844x; 1.1368x over previous
import functools

import jax
import jax.numpy as jnp
from jax import lax
from jax.experimental import pallas as pl
from jax.experimental.pallas import tpu as pltpu
from jax.experimental.pallas import tpu_sc as plsc

_D = 64
_BB = 128
_T = 200
_TPS = 2


@functools.cache
def _build(n_b: int, n_t: int, d: int):
    info = plsc.get_sparse_core_info()
    nc = info.num_cores
    nw = nc * info.num_subcores
    assert n_b == nw * _BB and n_t == _T and d == _D
    n_steps = _T // _TPS

    mesh = plsc.VectorSubcoreMesh(core_axis_name="c", subcore_axis_name="s")

    @functools.partial(
        pl.kernel,
        mesh=mesh,
        out_type=jax.ShapeDtypeStruct((_T, d // 8, n_b // 128, 8, 128),
                                      jnp.float32),
        compiler_params=pltpu.CompilerParams(
            use_tc_tiling_on_sc=False, needs_layout_passes=False
        ),
        scratch_types=[
            pltpu.VMEM((_T // _TPS, _TPS * _BB), jnp.int32),
            pltpu.VMEM((_TPS * _BB, d), jnp.float32),
            pltpu.VMEM((_TPS * _BB, d), jnp.float32),
            pltpu.VMEM((d, 129), jnp.float32),
            pltpu.VMEM((d, 129), jnp.float32),
            pltpu.VMEM((d, 129), jnp.float32),
            pltpu.VMEM((d, 129), jnp.float32),
            pltpu.SemaphoreType.DMA,
            pltpu.SemaphoreType.DMA,
            pltpu.SemaphoreType.DMA,
            pltpu.SemaphoreType.DMA,
            pltpu.SemaphoreType.DMA,
            pltpu.SemaphoreType.DMA,
        ],
    )
    def k(xp_hbm, table_hbm, out_hbm, idx_v, rows0, rows1,
          ta, tb, tc_, td, g0, g1, oa, ob, oc, od):
        wid = lax.axis_index("s") * nc + lax.axis_index("c")
        pltpu.sync_copy(xp_hbm.at[:, wid], idx_v)

        lanes = jnp.arange(16, dtype=jnp.int32)
        dvecs = [lanes + 16 * g for g in range(d // 16)]

        def fire_g(s, rows, gsem):
            pltpu.async_copy(table_hbm.at[idx_v.at[s]], rows, gsem)

        def wait_g(rows, gsem):
            pltpu.make_async_copy(
                table_hbm.at[pl.ds(0, _TPS * _BB)], rows, gsem
            ).wait()

        def transpose(rows, half, tr):
            def cbody(c, carry):
                cvec = jnp.full((16,), c, jnp.int32)
                for g in range(d // 16):
                    v = rows[half * _BB + c, pl.ds(16 * g, 16)]
                    plsc.store_scatter(tr, [dvecs[g], cvec], v)
                return carry

            lax.fori_loop(0, _BB, cbody, 0, unroll=4)

        def fire_o(t, tr, osem):
            for dd in range(d // 8):
                pltpu.async_copy(
                    tr.at[pl.ds(dd * 8, 8), pl.ds(0, 128)],
                    out_hbm.at[t, dd, wid],
                    osem,
                )

        def wait_o(tr, osem):
            for dd in range(d // 8):
                pltpu.make_async_copy(
                    tr.at[pl.ds(dd * 8, 8), pl.ds(0, 128)],
                    out_hbm.at[0, dd, wid],
                    osem,
                ).wait()

        def steppair(s, rows, bufs):
            (t_a, o_a), (t_b, o_b) = bufs
            transpose(rows, 0, t_a)
            fire_o(s * _TPS, t_a, o_a)
            transpose(rows, 1, t_b)
            fire_o(s * _TPS + 1, t_b, o_b)

        bufs0 = ((ta, oa), (tb, ob))
        bufs1 = ((tc_, oc), (td, od))

        fire_g(0, rows0, g0)
        fire_g(1, rows1, g1)

        def body(i, carry):
            s0 = 2 * i

            wait_g(rows0, g0)

            @pl.when(i > 0)
            def _():
                for tr, osem in bufs0:
                    wait_o(tr, osem)

            steppair(s0, rows0, bufs0)
            fire_g(s0 + 2, rows0, g0)

            wait_g(rows1, g1)

            @pl.when(i > 0)
            def _():
                for tr, osem in bufs1:
                    wait_o(tr, osem)

            steppair(s0 + 1, rows1, bufs1)
            fire_g(s0 + 3, rows1, g1)
            return carry

        lax.fori_loop(0, n_steps // 2 - 1, body, 0)

        wait_g(rows0, g0)
        for tr, osem in bufs0:
            wait_o(tr, osem)
        steppair(n_steps - 2, rows0, bufs0)
        wait_g(rows1, g1)
        for tr, osem in bufs1:
            wait_o(tr, osem)
        steppair(n_steps - 1, rows1, bufs1)
        for tr, osem in (*bufs0, *bufs1):
            wait_o(tr, osem)

    return k


def kernel(x, table):
    s0, s1 = x.shape
    nw = s0 // _BB
    xp = (x.reshape(nw, _BB, s1 // _TPS, _TPS)
          .transpose(2, 0, 3, 1)
          .reshape(s1 // _TPS, nw, _TPS * _BB))
    out5 = _build(s0, s1, _D)(xp, table)
    return out5.transpose(2, 4, 0, 1, 3).reshape(s0, s1, _D)


# speedup vs baseline: 1.9926x; 1.0041x over previous
import functools

import jax
import jax.numpy as jnp
from jax import lax
from jax.experimental import pallas as pl
from jax.experimental.pallas import tpu as pltpu
from jax.experimental.pallas import tpu_sc as plsc

_D = 64
_BB = 128
_T = 200
_TPS = 2


@functools.cache
def _build(n_b: int, n_t: int, d: int):
    info = plsc.get_sparse_core_info()
    nc = info.num_cores
    nw = nc * info.num_subcores
    assert n_b == nw * _BB and n_t == _T and d == _D
    n_steps = _T // _TPS

    mesh = plsc.VectorSubcoreMesh(core_axis_name="c", subcore_axis_name="s")

    @functools.partial(
        pl.kernel,
        mesh=mesh,
        out_type=jax.ShapeDtypeStruct((_T, d // 8, n_b // 128, 8, 128),
                                      jnp.float32),
        compiler_params=pltpu.CompilerParams(
            use_tc_tiling_on_sc=False, needs_layout_passes=False
        ),
        scratch_types=[
            pltpu.VMEM((_T // _TPS, _TPS * _BB), jnp.int32),
            pltpu.VMEM((_TPS * _BB, d), jnp.float32),
            pltpu.VMEM((_TPS * _BB, d), jnp.float32),
            pltpu.VMEM((d, 129), jnp.float32),
            pltpu.VMEM((d, 129), jnp.float32),
            pltpu.VMEM((d, 129), jnp.float32),
            pltpu.VMEM((d, 129), jnp.float32),
            pltpu.SemaphoreType.DMA,
            pltpu.SemaphoreType.DMA,
            pltpu.SemaphoreType.DMA,
            pltpu.SemaphoreType.DMA,
            pltpu.SemaphoreType.DMA,
            pltpu.SemaphoreType.DMA,
        ],
    )
    def k(xp_hbm, table_hbm, out_hbm, idx_v, rows0, rows1,
          ta, tb, tc_, td, g0, g1, oa, ob, oc, od):
        wid = lax.axis_index("s") * nc + lax.axis_index("c")
        pltpu.sync_copy(xp_hbm.at[:, wid], idx_v)

        lanes = jnp.arange(16, dtype=jnp.int32)
        dvecs = [lanes + 16 * g for g in range(d // 16)]

        def fire_g(s, rows, gsem):
            pltpu.async_copy(table_hbm.at[idx_v.at[s]], rows, gsem)

        def wait_g(rows, gsem):
            pltpu.make_async_copy(
                table_hbm.at[pl.ds(0, _TPS * _BB)], rows, gsem
            ).wait()

        def transpose(rows, half, tr):
            def cbody(c, carry):
                cvec = jnp.full((16,), c, jnp.int32)
                for g in range(d // 16):
                    v = rows[half * _BB + c, pl.ds(16 * g, 16)]
                    plsc.store_scatter(tr, [dvecs[g], cvec], v)
                return carry

            lax.fori_loop(0, _BB, cbody, 0, unroll=8)

        def fire_o(t, tr, osem):
            for dd in range(d // 8):
                pltpu.async_copy(
                    tr.at[pl.ds(dd * 8, 8), pl.ds(0, 128)],
                    out_hbm.at[t, dd, wid],
                    osem,
                )

        def wait_o(tr, osem):
            for dd in range(d // 8):
                pltpu.make_async_copy(
                    tr.at[pl.ds(dd * 8, 8), pl.ds(0, 128)],
                    out_hbm.at[0, dd, wid],
                    osem,
                ).wait()

        def steppair(s, rows, bufs):
            (t_a, o_a), (t_b, o_b) = bufs
            transpose(rows, 0, t_a)
            fire_o(s * _TPS, t_a, o_a)
            transpose(rows, 1, t_b)
            fire_o(s * _TPS + 1, t_b, o_b)

        bufs0 = ((ta, oa), (tb, ob))
        bufs1 = ((tc_, oc), (td, od))

        fire_g(0, rows0, g0)
        fire_g(1, rows1, g1)

        def body(i, carry):
            s0 = 2 * i

            wait_g(rows0, g0)

            @pl.when(i > 0)
            def _():
                for tr, osem in bufs0:
                    wait_o(tr, osem)

            steppair(s0, rows0, bufs0)
            fire_g(s0 + 2, rows0, g0)

            wait_g(rows1, g1)

            @pl.when(i > 0)
            def _():
                for tr, osem in bufs1:
                    wait_o(tr, osem)

            steppair(s0 + 1, rows1, bufs1)
            fire_g(s0 + 3, rows1, g1)
            return carry

        lax.fori_loop(0, n_steps // 2 - 1, body, 0)

        wait_g(rows0, g0)
        for tr, osem in bufs0:
            wait_o(tr, osem)
        steppair(n_steps - 2, rows0, bufs0)
        wait_g(rows1, g1)
        for tr, osem in bufs1:
            wait_o(tr, osem)
        steppair(n_steps - 1, rows1, bufs1)
        for tr, osem in (*bufs0, *bufs1):
            wait_o(tr, osem)

    return k


def kernel(x, table):
    s0, s1 = x.shape
    nw = s0 // _BB
    xp = (x.reshape(nw, _BB, s1 // _TPS, _TPS)
          .transpose(2, 0, 3, 1)
          .reshape(s1 // _TPS, nw, _TPS * _BB))
    out5 = _build(s0, s1, _D)(xp, table)
    return out5.transpose(2, 4, 0, 1, 3).reshape(s0, s1, _D)
